# baseline (speedup 1.0000x reference)
import functools

import jax
import jax.numpy as jnp
from jax import lax
from jax.experimental import pallas as pl
from jax.experimental.pallas import tpu as pltpu
from jax.experimental.pallas import tpu_sc as plsc

BATCH = 16384
HIST = 50
EMBED = 32
VOCAB = 1000000
PADW = 128
N = BATCH * HIST
NC, NS = 2, 16
NW = NC * NS
GRP = 128
NBLK = BATCH // GRP
NGRP = HIST * NBLK
G = NGRP // NW
R = 8
NCH = G // R
NPAIR = (NCH - 1) // 2

_MESH = plsc.VectorSubcoreMesh(
    core_axis_name="c", subcore_axis_name="s", num_cores=NC, num_subcores=NS
)


_LW = 4096
_LGRID = (VOCAB + _LW - 1) // _LW


def _eye(n):
    r = lax.broadcasted_iota(jnp.int32, (n, n), 0)
    c = lax.broadcasted_iota(jnp.int32, (n, n), 1)
    return (r == c).astype(jnp.float32)


def _linearize_body(t_ref, o_ref):
    x = t_ref[...]
    xt = lax.dot_general(
        x, _eye(EMBED), (((0,), (0,)), ((), ())),
        preferred_element_type=jnp.float32,
        precision=lax.Precision.HIGHEST,
    )
    x3 = xt.reshape(_LW // 4, 4, EMBED)
    o_ref[...] = jnp.concatenate([x3[:, t, :] for t in range(4)], axis=1)


def _linearize(table_t):
    return pl.pallas_call(
        _linearize_body,
        grid=(_LGRID,),
        in_specs=[pl.BlockSpec((EMBED, _LW), lambda i: (0, i))],
        out_specs=pl.BlockSpec((_LW // 4, 4 * EMBED), lambda i: (i, 0)),
        out_shape=jax.ShapeDtypeStruct((VOCAB // 4, 4 * EMBED), jnp.float32),
    )(table_t)


@functools.partial(
    pl.kernel,
    out_type=jax.ShapeDtypeStruct((HIST, BATCH, PADW), jnp.float32),
    mesh=_MESH,
    scratch_types=[
        pltpu.VMEM((G, GRP), jnp.int32),
        pltpu.VMEM((2, R, GRP, EMBED), jnp.float32),
        pltpu.SemaphoreType.DMA,
        pltpu.SemaphoreType.DMA,
        pltpu.SemaphoreType.DMA,
        pltpu.SemaphoreType.DMA,
    ],
    compiler_params=pltpu.CompilerParams(use_tc_tiling_on_sc=False),
)
def _sc_gather(idx_hbm, table_hbm, out_hbm, idx_v, bufs, gsa, gsb, wsa, wsb):
    wid = lax.axis_index("s") * NC + lax.axis_index("c")
    pltpu.sync_copy(idx_hbm.at[wid], idx_v)

    def fire_g(c, half, sem):
        for b in range(R):
            pltpu.async_copy(
                table_hbm.at[idx_v.at[c * R + b]], bufs.at[half, b], sem
            )

    def drain_g(half, sem):
        for b in range(R):
            pltpu.make_async_copy(
                table_hbm.at[idx_v.at[0]], bufs.at[half, b], sem
            ).wait()

    def _dst(c, b):
        gid = wid * G + c * R + b
        h = gid // NBLK
        b0 = (gid % NBLK) * GRP
        return out_hbm.at[h, pl.ds(b0, GRP), pl.ds(0, EMBED)]

    def fire_w(c, half, sem):
        for b in range(R):
            pltpu.async_copy(bufs.at[half, b], _dst(c, b), sem)

    def drain_w(half, sem):
        for b in range(R):
            pltpu.make_async_copy(bufs.at[half, b], _dst(0, b), sem).wait()

    fire_g(0, 0, gsa)

    def body(s, carry):
        c0 = 2 * s

        @pl.when(s > 0)
        def _():
            drain_w(1, wsb)

        fire_g(c0 + 1, 1, gsb)
        drain_g(0, gsa)
        fire_w(c0, 0, wsa)
        drain_w(0, wsa)
        fire_g(c0 + 2, 0, gsa)
        drain_g(1, gsb)
        fire_w(c0 + 1, 1, wsb)
        return carry

    lax.fori_loop(0, NPAIR, body, 0)

    drain_w(1, wsb)
    drain_g(0, gsa)
    fire_w(NCH - 1, 0, wsa)
    drain_w(0, wsa)


_OB = 2048
_OGRID = (HIST, BATCH // _OB)


def _out_body(g_ref, o_ref):
    x = g_ref[0]
    o_ref[0] = jnp.transpose(x[:, :EMBED])


def _slice_transpose(g3p):
    return pl.pallas_call(
        _out_body,
        grid=_OGRID,
        in_specs=[pl.BlockSpec((1, _OB, PADW), lambda h, j: (h, j, 0))],
        out_specs=pl.BlockSpec((1, EMBED, _OB), lambda h, j: (h, 0, j)),
        out_shape=jax.ShapeDtypeStruct((HIST, EMBED, BATCH), jnp.float32),
    )(g3p)


def kernel(indices, table):
    tbl = _linearize(table.T).reshape(VOCAB, EMBED)
    idx = indices.T.reshape(NW, G, GRP)
    g3p = _sc_gather(idx, tbl)
    out3 = _slice_transpose(g3p)
    return jnp.transpose(out3, (2, 0, 1))


# speedup vs baseline: 1.3635x; 1.3635x over previous
import functools

import jax
import jax.numpy as jnp
from jax import lax
from jax.experimental import pallas as pl
from jax.experimental.pallas import tpu as pltpu
from jax.experimental.pallas import tpu_sc as plsc

BATCH = 16384
HIST = 50
EMBED = 32
VOCAB = 1000000
PADW = 128
N = BATCH * HIST
NC, NS = 2, 16
NW = NC * NS
GRP = 128
NBLK = BATCH // GRP
NGRP = HIST * NBLK
G = NGRP // NW
R = 8
NCH = G // R
NPAIR = (NCH - 1) // 2

_MESH = plsc.VectorSubcoreMesh(
    core_axis_name="c", subcore_axis_name="s", num_cores=NC, num_subcores=NS
)


_LW = 4096
_LGRID = (VOCAB + _LW - 1) // _LW


def _linearize_body(t_ref, o_ref):
    xt = jnp.transpose(t_ref[...])
    x3 = xt.reshape(_LW // 4, 4, EMBED)
    o_ref[...] = jnp.concatenate([x3[:, t, :] for t in range(4)], axis=1)


def _linearize(table_t):
    return pl.pallas_call(
        _linearize_body,
        grid=(_LGRID,),
        in_specs=[pl.BlockSpec((EMBED, _LW), lambda i: (0, i))],
        out_specs=pl.BlockSpec((_LW // 4, 4 * EMBED), lambda i: (i, 0)),
        out_shape=jax.ShapeDtypeStruct((VOCAB // 4, 4 * EMBED), jnp.float32),
    )(table_t)


@functools.partial(
    pl.kernel,
    out_type=jax.ShapeDtypeStruct((HIST, BATCH, PADW), jnp.float32),
    mesh=_MESH,
    scratch_types=[
        pltpu.VMEM((G, GRP), jnp.int32),
        pltpu.VMEM((2, R, GRP, EMBED), jnp.float32),
        pltpu.SemaphoreType.DMA,
        pltpu.SemaphoreType.DMA,
        pltpu.SemaphoreType.DMA,
        pltpu.SemaphoreType.DMA,
    ],
    compiler_params=pltpu.CompilerParams(use_tc_tiling_on_sc=False),
)
def _sc_gather(idx_hbm, table_hbm, out_hbm, idx_v, bufs, gsa, gsb, wsa, wsb):
    wid = lax.axis_index("s") * NC + lax.axis_index("c")
    pltpu.sync_copy(idx_hbm.at[wid], idx_v)

    def fire_g(c, half, sem):
        for b in range(R):
            pltpu.async_copy(
                table_hbm.at[idx_v.at[c * R + b]], bufs.at[half, b], sem
            )

    def drain_g(half, sem):
        for b in range(R):
            pltpu.make_async_copy(
                table_hbm.at[idx_v.at[0]], bufs.at[half, b], sem
            ).wait()

    def _dst(c, b):
        gid = wid * G + c * R + b
        h = gid // NBLK
        b0 = (gid % NBLK) * GRP
        return out_hbm.at[h, pl.ds(b0, GRP), pl.ds(0, EMBED)]

    def fire_w(c, half, sem):
        for b in range(R):
            pltpu.async_copy(bufs.at[half, b], _dst(c, b), sem)

    def drain_w(half, sem):
        for b in range(R):
            pltpu.make_async_copy(bufs.at[half, b], _dst(0, b), sem).wait()

    fire_g(0, 0, gsa)

    def body(s, carry):
        c0 = 2 * s

        @pl.when(s > 0)
        def _():
            drain_w(1, wsb)

        fire_g(c0 + 1, 1, gsb)
        drain_g(0, gsa)
        fire_w(c0, 0, wsa)
        drain_w(0, wsa)
        fire_g(c0 + 2, 0, gsa)
        drain_g(1, gsb)
        fire_w(c0 + 1, 1, wsb)
        return carry

    lax.fori_loop(0, NPAIR, body, 0)

    drain_w(1, wsb)
    drain_g(0, gsa)
    fire_w(NCH - 1, 0, wsa)
    drain_w(0, wsa)


_OB = 2048
_OGRID = (HIST, BATCH // _OB)


def _out_body(g_ref, o_ref):
    x = g_ref[0]
    o_ref[0] = jnp.transpose(x[:, :EMBED])


def _slice_transpose(g3p):
    return pl.pallas_call(
        _out_body,
        grid=_OGRID,
        in_specs=[pl.BlockSpec((1, _OB, PADW), lambda h, j: (h, j, 0))],
        out_specs=pl.BlockSpec((1, EMBED, _OB), lambda h, j: (h, 0, j)),
        out_shape=jax.ShapeDtypeStruct((HIST, EMBED, BATCH), jnp.float32),
    )(g3p)


def kernel(indices, table):
    tbl = _linearize(table.T).reshape(VOCAB, EMBED)
    idx = indices.T.reshape(NW, G, GRP)
    g3p = _sc_gather(idx, tbl)
    out3 = _slice_transpose(g3p)
    return jnp.transpose(out3, (2, 0, 1))
